# baseline (speedup 1.0000x reference)
import functools

import jax
import jax.numpy as jnp
from jax import lax
from jax.experimental import pallas as pl
from jax.experimental.pallas import tpu as pltpu
from jax.experimental.pallas import tpu_sc as plsc

N_NODES = 100000
DIM_MEM = 128
DIM_MSG = 272
B = 100000

NC, NS = 2, 16
NW = NC * NS
BPW = 3200
LAST_BPW = B - (NW - 1) * BPW
CHUNK = 80
NCHUNK = BPW // CHUNK
NBUF = 2
LANES = 16
LU_PER_CHUNK = BPW // LANES // NCHUNK

ROWS_BLK = 3200
GRID = -(-N_NODES // ROWS_BLK)


def _gru_body(mail_ref, mem_ref, wih_ref, whh_ref, bih_ref, bhh_ref, out_ref):
    xt = mail_ref[...]
    h = mem_ref[...]
    gi = lax.dot_general(xt.astype(jnp.bfloat16),
                         wih_ref[...].astype(jnp.bfloat16),
                         (((0,), (0,)), ((), ())),
                         preferred_element_type=jnp.float32) + bih_ref[...][None, :]
    gh = lax.dot_general(h.astype(jnp.bfloat16),
                         whh_ref[...].astype(jnp.bfloat16),
                         (((1,), (1,)), ((), ())),
                         preferred_element_type=jnp.float32) + bhh_ref[...][None, :]
    r = jax.nn.sigmoid(gi[:, :DIM_MEM] + gh[:, :DIM_MEM])
    z = jax.nn.sigmoid(gi[:, DIM_MEM:2 * DIM_MEM] + gh[:, DIM_MEM:2 * DIM_MEM])
    n = jnp.tanh(gi[:, 2 * DIM_MEM:] + r * gh[:, 2 * DIM_MEM:])
    out_ref[...] = (1.0 - z) * n + z * h


_gru_all = pl.pallas_call(
    _gru_body,
    grid=(GRID,),
    in_specs=[
        pl.BlockSpec((DIM_MSG, ROWS_BLK), lambda i: (0, i)),
        pl.BlockSpec((ROWS_BLK, DIM_MEM), lambda i: (i, 0)),
        pl.BlockSpec((DIM_MSG, 3 * DIM_MEM), lambda i: (0, 0)),
        pl.BlockSpec((3 * DIM_MEM, DIM_MEM), lambda i: (0, 0)),
        pl.BlockSpec((3 * DIM_MEM,), lambda i: (0,)),
        pl.BlockSpec((3 * DIM_MEM,), lambda i: (0,)),
    ],
    out_specs=pl.BlockSpec((ROWS_BLK, DIM_MEM), lambda i: (i, 0)),
    out_shape=jax.ShapeDtypeStruct((N_NODES, DIM_MEM), jnp.float32),
)


def _sc_body(nodes_hbm, newmem_hbm, lu_hbm, out1_hbm, out2_hbm,
             idx_v, rows_v, tab_v, luout_v, *sems):
    g_sem = sems[0:NBUF]
    o_sem = sems[NBUF:2 * NBUF]
    t_sem = sems[2 * NBUF]

    wid = lax.axis_index("s") * NC + lax.axis_index("c")
    base = wid * BPW
    is_last = wid == NW - 1

    @pl.when(jnp.logical_not(is_last))
    def _():
        pltpu.sync_copy(nodes_hbm.at[pl.ds(base, BPW)], idx_v)

    @pl.when(is_last)
    def _():
        pltpu.sync_copy(nodes_hbm.at[pl.ds(base, LAST_BPW)],
                        idx_v.at[pl.ds(0, LAST_BPW)])

    tab_copy = pltpu.make_async_copy(lu_hbm, tab_v, t_sem)
    tab_copy.start()

    def pred(i):
        return base + (i + 1) * CHUNK <= B

    def mk(i):
        b = i % NBUF
        idx_sl = idx_v.at[pl.ds(i * CHUNK, CHUNK)]
        gc = pltpu.make_async_copy(newmem_hbm.at[idx_sl], rows_v.at[b],
                                   g_sem[b])
        oc = pltpu.make_async_copy(rows_v.at[b],
                                   out1_hbm.at[pl.ds(base + i * CHUNK, CHUNK)],
                                   o_sem[b])
        return gc, oc

    def lu_steps(i):
        for k in range(i * LU_PER_CHUNK, (i + 1) * LU_PER_CHUNK):
            off = k * LANES
            iv = jnp.clip(idx_v[pl.ds(off, LANES)], 0, N_NODES - 1)
            luout_v[pl.ds(off, LANES)] = plsc.load_gather(tab_v, [iv])

    g = [None] * NBUF
    oc = [None] * NBUF
    for i in range(NCHUNK + 1):
        if i < NCHUNK:
            b = i % NBUF
            if oc[b] is not None:
                c, p = oc[b]
                pl.when(p)(c.wait)
                oc[b] = None
            gc, ocopy = mk(i)
            p = pred(i)
            pl.when(p)(gc.start)
            g[b] = (gc, ocopy, p)
        if i == 0:
            tab_copy.wait()
        if i >= 1:
            lu_steps(i - 1)
            bj = (i - 1) % NBUF
            gc, ocopy, p = g[bj]
            pl.when(p)(gc.wait)
            pl.when(p)(ocopy.start)
            oc[bj] = (ocopy, p)
    for b in range(NBUF):
        if oc[b] is not None:
            c, p = oc[b]
            pl.when(p)(c.wait)

    @pl.when(jnp.logical_not(is_last))
    def _():
        pltpu.sync_copy(luout_v, out2_hbm.at[pl.ds(base, BPW)])

    @pl.when(is_last)
    def _():
        pltpu.sync_copy(luout_v.at[pl.ds(0, LAST_BPW)],
                        out2_hbm.at[pl.ds(base, LAST_BPW)])


@functools.cache
def _sc_gather():
    mesh = plsc.VectorSubcoreMesh(core_axis_name="c", subcore_axis_name="s",
                                  num_cores=NC, num_subcores=NS)
    return functools.partial(
        pl.kernel,
        out_type=(jax.ShapeDtypeStruct((B, DIM_MEM), jnp.float32),
                  jax.ShapeDtypeStruct((B,), jnp.float32)),
        mesh=mesh,
        name="sc_gather",
        compiler_params=pltpu.CompilerParams(needs_layout_passes=False),
        scratch_types=(
            [pltpu.VMEM((BPW,), jnp.int32),
             pltpu.VMEM((NBUF, CHUNK, DIM_MEM), jnp.float32),
             pltpu.VMEM((N_NODES,), jnp.float32),
             pltpu.VMEM((BPW,), jnp.float32)]
            + [pltpu.SemaphoreType.DMA] * (2 * NBUF + 1)
        ),
    )(_sc_body)


def kernel(nodes, memory, mailbox, last_update, W_ih, W_hh, b_ih, b_hh):
    nodes32 = nodes.astype(jnp.int32)
    new_mem = _gru_all(mailbox.T, memory, W_ih.T, W_hh, b_ih, b_hh)
    out1, out2 = _sc_gather()(nodes32, new_mem, last_update)
    return out1, out2
